# baseline (speedup 1.0000x reference)
import jax
import jax.numpy as jnp
from jax import lax
from jax.experimental import pallas as pl
from jax.experimental.pallas import tpu as pltpu
from jax.experimental.pallas import tpu_sc as plsc

LANES = 16
NBIN = 65
IDX_CHUNK = 128


def _build_sc_lookup(L, D):
  info = plsc.get_sparse_core_info()
  nc, ns = info.num_cores, info.num_subcores
  nw = nc * ns
  rpw = L // nw
  n_chunk = L // IDX_CHUNK
  n_vec = IDX_CHUNK // LANES

  mesh = plsc.VectorSubcoreMesh(core_axis_name="c", subcore_axis_name="s")

  def body(idx_hbm, table_hbm, out_hbm, idx_v, ib_v, slab, sem):
    wid = lax.axis_index("s") * nc + lax.axis_index("c")
    base = wid * rpw
    pltpu.sync_copy(idx_hbm, idx_v.at[pl.ds(0, L)])

    def row_step(r, carry):
      i = base + r
      s = idx_v[pl.ds(i, LANES)][0]

      def chunk_step(c, carry2):
        def vec_step(v, carry3):
          d = idx_v[pl.ds(c * IDX_CHUNK + v * LANES, LANES)] - s + 32
          ib_v[c, pl.ds(v * LANES, LANES)] = jnp.clip(d, 0, NBIN - 1)
          return carry3

        return lax.fori_loop(jnp.int32(0), jnp.int32(n_vec), vec_step, carry2)

      lax.fori_loop(jnp.int32(0), jnp.int32(n_chunk), chunk_step, jnp.int32(0))

      cps = [
          pltpu.async_copy(
              table_hbm.at[ib_v.at[jnp.int32(k)]],
              slab.at[pl.ds(jnp.int32(k * IDX_CHUNK), IDX_CHUNK)],
              sem,
          )
          for k in range(n_chunk)
      ]
      for cp in cps:
        cp.wait()
      pltpu.sync_copy(slab, out_hbm.at[i])
      return carry

    lax.fori_loop(jnp.int32(0), jnp.int32(rpw), row_step, jnp.int32(0))

  return pl.kernel(
      body,
      mesh=mesh,
      compiler_params=pltpu.CompilerParams(use_tc_tiling_on_sc=False),
      out_type=jax.ShapeDtypeStruct((L, L, D), jnp.float32),
      scratch_types=[
          pltpu.VMEM((L + LANES,), jnp.int32),
          pltpu.VMEM((n_chunk, IDX_CHUNK), jnp.int32),
          pltpu.VMEM((L, D), jnp.float32),
          pltpu.SemaphoreType.DMA,
      ],
  )


def kernel(idx, stride, emb_weight):
  B, L = idx.shape
  D = emb_weight.shape[-1]
  idx32 = idx.reshape(L).astype(jnp.int32)
  table = emb_weight.astype(jnp.float32)
  out = _build_sc_lookup(L, D)(idx32, table)
  return out.reshape(B, L, L, D)


# speedup vs baseline: 5.4175x; 5.4175x over previous
import jax
import jax.numpy as jnp
from jax import lax
from jax.experimental import pallas as pl
from jax.experimental.pallas import tpu as pltpu
from jax.experimental.pallas import tpu_sc as plsc

LANES = 16
NBIN = 65


def _build_sc_lookup(L, D):
  info = plsc.get_sparse_core_info()
  nc, ns = info.num_cores, info.num_subcores
  nw = nc * ns
  rpw = L // nw
  half = L // 2
  unit = half * D
  n_grp = half // LANES

  mesh = plsc.VectorSubcoreMesh(core_axis_name="c", subcore_axis_name="s")

  def body(idx_hbm, table_hbm, out_hbm, idx_v, table_v, slab, sem0, sem1):
    wid = lax.axis_index("s") * nc + lax.axis_index("c")
    base = wid * rpw
    pltpu.sync_copy(idx_hbm, idx_v.at[pl.ds(0, L)])
    pltpu.sync_copy(table_hbm, table_v)
    iota_d = lax.iota(jnp.int32, LANES) * D
    sems = (sem0, sem1)

    def row_step(r, carry):
      i = base + r
      s = idx_v[pl.ds(i, LANES)][0]
      for h in range(2):
        slab_h = slab.at[jnp.int32(h)]

        @pl.when(r >= 1)
        def _drain(h=h, slab_h=slab_h):
          pltpu.make_async_copy(
              slab_h, out_hbm.at[pl.ds(0, unit)], sems[h]).wait()

        def grp_step(g, carry2, h=h, slab_h=slab_h):
          jv = idx_v[pl.ds(h * half + g * LANES, LANES)]
          jb = jnp.clip(jv - s + 32, 0, NBIN - 1)
          jb_d = jb * D
          fbase = g * (LANES * D) + iota_d
          for d in range(D):
            vals = plsc.load_gather(table_v, [jb_d + d])
            plsc.store_scatter(slab_h, [fbase + d], vals)
          return carry2

        lax.fori_loop(jnp.int32(0), jnp.int32(n_grp), grp_step, jnp.int32(0))
        off = i * (L * D) + h * unit
        pltpu.async_copy(slab_h, out_hbm.at[pl.ds(off, unit)], sems[h])
      return carry

    lax.fori_loop(jnp.int32(0), jnp.int32(rpw), row_step, jnp.int32(0))
    for h in range(2):
      pltpu.make_async_copy(
          slab.at[jnp.int32(h)], out_hbm.at[pl.ds(0, unit)], sems[h]).wait()

  return pl.kernel(
      body,
      mesh=mesh,
      compiler_params=pltpu.CompilerParams(
          use_tc_tiling_on_sc=False, needs_layout_passes=False),
      out_type=jax.ShapeDtypeStruct((L * L * D,), jnp.float32),
      scratch_types=[
          pltpu.VMEM((L + LANES,), jnp.int32),
          pltpu.VMEM((NBIN * D,), jnp.float32),
          pltpu.VMEM((2, unit), jnp.float32),
          pltpu.SemaphoreType.DMA,
          pltpu.SemaphoreType.DMA,
      ],
  )


def kernel(idx, stride, emb_weight):
  B, L = idx.shape
  D = emb_weight.shape[-1]
  idx32 = idx.reshape(L).astype(jnp.int32)
  table = emb_weight.astype(jnp.float32).reshape(NBIN * D)
  out = _build_sc_lookup(L, D)(idx32, table)
  return out.reshape(B, L, L, D)


# speedup vs baseline: 9.8400x; 1.8164x over previous
import jax
import jax.numpy as jnp
from jax import lax
from jax.experimental import pallas as pl
from jax.experimental.pallas import tpu as pltpu
from jax.experimental.pallas import tpu_sc as plsc

LANES = 16
NBIN = 65


def _build_sc_lookup(L, D):
  info = plsc.get_sparse_core_info()
  nc, ns = info.num_cores, info.num_subcores
  nw = nc * ns
  rpw = L // nw
  half = L // 2
  unit = half * D
  n_grp = half // LANES

  mesh = plsc.VectorSubcoreMesh(core_axis_name="c", subcore_axis_name="s")

  def body(idx_hbm, table_hbm, out_hbm, idx_v, table_v, slab, sem0, sem1):
    wid = lax.axis_index("s") * nc + lax.axis_index("c")
    base = wid * rpw
    pltpu.sync_copy(idx_hbm, idx_v.at[pl.ds(0, L)])
    pltpu.sync_copy(table_hbm, table_v)
    sems = (sem0, sem1)

    def row_step(r, carry):
      i = base + r
      s = idx_v[pl.ds(i, LANES)][0]
      for h in range(2):
        slab_h = slab.at[jnp.int32(h)]

        @pl.when(r >= 1)
        def _drain(h=h, slab_h=slab_h):
          pltpu.make_async_copy(
              slab_h, out_hbm.at[pl.ds(0, unit)], sems[h]).wait()

        def grp_step(g, carry2, h=h, slab_h=slab_h):
          jv = idx_v[pl.ds(h * half + g * LANES, LANES)]
          jb = jnp.clip(jv - s + 32, 0, NBIN - 1)
          jb_d = jb * D
          gbase = g * (LANES * D)
          for k in range(LANES):
            a = jb_d[k]
            for t in range(D // LANES):
              slab_h[pl.ds(gbase + k * D + t * LANES, LANES)] = (
                  table_v[pl.ds(a + t * LANES, LANES)])
          return carry2

        lax.fori_loop(jnp.int32(0), jnp.int32(n_grp), grp_step, jnp.int32(0))
        off = i * (L * D) + h * unit
        pltpu.async_copy(slab_h, out_hbm.at[pl.ds(off, unit)], sems[h])
      return carry

    lax.fori_loop(jnp.int32(0), jnp.int32(rpw), row_step, jnp.int32(0))
    for h in range(2):
      pltpu.make_async_copy(
          slab.at[jnp.int32(h)], out_hbm.at[pl.ds(0, unit)], sems[h]).wait()

  return pl.kernel(
      body,
      mesh=mesh,
      compiler_params=pltpu.CompilerParams(
          use_tc_tiling_on_sc=False, needs_layout_passes=False),
      out_type=jax.ShapeDtypeStruct((L * L * D,), jnp.float32),
      scratch_types=[
          pltpu.VMEM((L + LANES,), jnp.int32),
          pltpu.VMEM((NBIN * D,), jnp.float32),
          pltpu.VMEM((2, unit), jnp.float32),
          pltpu.SemaphoreType.DMA,
          pltpu.SemaphoreType.DMA,
      ],
  )


def kernel(idx, stride, emb_weight):
  B, L = idx.shape
  D = emb_weight.shape[-1]
  idx32 = idx.reshape(L).astype(jnp.int32)
  table = emb_weight.astype(jnp.float32).reshape(NBIN * D)
  out = _build_sc_lookup(L, D)(idx32, table)
  return out.reshape(B, L, L, D)


# speedup vs baseline: 13.6476x; 1.3870x over previous
import jax
import jax.numpy as jnp
from jax import lax
from jax.experimental import pallas as pl
from jax.experimental.pallas import tpu as pltpu
from jax.experimental.pallas import tpu_sc as plsc

LANES = 16
NBIN = 65


def _build_sc_lookup(L, D):
  info = plsc.get_sparse_core_info()
  nc, ns = info.num_cores, info.num_subcores
  nw = nc * ns
  rpw = L // nw
  half = L // 2
  unit = half * D
  n_grp = half // LANES

  mesh = plsc.VectorSubcoreMesh(core_axis_name="c", subcore_axis_name="s")

  def body(idx_hbm, table_hbm, out_hbm, idx_v, table_v, slab, sem0, sem1):
    wid = lax.axis_index("s") * nc + lax.axis_index("c")
    base = wid * rpw
    pltpu.sync_copy(idx_hbm, idx_v.at[pl.ds(0, L)])
    pltpu.sync_copy(table_hbm, table_v)
    sems = (sem0, sem1)

    def row_step(r, carry):
      i = base + r
      s = idx_v[pl.ds(i, LANES)][0]
      for h in range(2):
        slab_h = slab.at[jnp.int32(h)]

        @pl.when(r >= 1)
        def _drain(h=h, slab_h=slab_h):
          pltpu.make_async_copy(
              slab_h, out_hbm.at[pl.ds(0, unit)], sems[h]).wait()

        def grp_step(g, carry2, h=h, slab_h=slab_h):
          jv = idx_v[pl.ds(h * half + g * LANES, LANES)]
          jb = jnp.clip(jv - s + 32, 0, NBIN - 1)
          jb_d = jb * D
          gbase = g * (LANES * D)
          nt = D // LANES
          for k0 in range(0, LANES, 8):
            addrs = [jb_d[k0 + m] for m in range(8)]
            vals = [table_v[pl.ds(addrs[m] + t * LANES, LANES)]
                    for m in range(8) for t in range(nt)]
            for m in range(8):
              for t in range(nt):
                slab_h[pl.ds(gbase + (k0 + m) * D + t * LANES, LANES)] = (
                    vals[m * nt + t])
          return carry2

        lax.fori_loop(jnp.int32(0), jnp.int32(n_grp), grp_step, jnp.int32(0))
        off = i * (L * D) + h * unit
        pltpu.async_copy(slab_h, out_hbm.at[pl.ds(off, unit)], sems[h])
      return carry

    lax.fori_loop(jnp.int32(0), jnp.int32(rpw), row_step, jnp.int32(0))
    for h in range(2):
      pltpu.make_async_copy(
          slab.at[jnp.int32(h)], out_hbm.at[pl.ds(0, unit)], sems[h]).wait()

  return pl.kernel(
      body,
      mesh=mesh,
      compiler_params=pltpu.CompilerParams(
          use_tc_tiling_on_sc=False, needs_layout_passes=False),
      out_type=jax.ShapeDtypeStruct((L * L * D,), jnp.float32),
      scratch_types=[
          pltpu.VMEM((L + LANES,), jnp.int32),
          pltpu.VMEM((NBIN * D,), jnp.float32),
          pltpu.VMEM((2, unit), jnp.float32),
          pltpu.SemaphoreType.DMA,
          pltpu.SemaphoreType.DMA,
      ],
  )


def kernel(idx, stride, emb_weight):
  B, L = idx.shape
  D = emb_weight.shape[-1]
  idx32 = idx.reshape(L).astype(jnp.int32)
  table = emb_weight.astype(jnp.float32).reshape(NBIN * D)
  out = _build_sc_lookup(L, D)(idx32, table)
  return out.reshape(B, L, L, D)
